# baseline (speedup 1.0000x reference)
import functools

import jax
import jax.numpy as jnp
from jax import lax
from jax.experimental import pallas as pl
from jax.experimental.pallas import tpu as pltpu
from jax.experimental.pallas import tpu_sc as plsc

T = 8192
B = 4
D = 768
NC = 2
NS = 16
NW = NC * NS
ROWS_PER_W = T // NW
C = 8
NCHUNK = ROWS_PER_W // C
LANES = 16
G = D // LANES

_mesh = plsc.VectorSubcoreMesh(core_axis_name="c", subcore_axis_name="s")


@functools.partial(
    pl.kernel,
    mesh=_mesh,
    out_type=jax.ShapeDtypeStruct((T * B, D), jnp.float32),
    scratch_types=[
        pltpu.VMEM((C * B, D), jnp.float32),
        pltpu.VMEM((C, D), jnp.float32),
    ],
)
def _pos_add(x_hbm, pos_hbm, out_hbm, xbuf, pbuf):
    wid = lax.axis_index("s") * NC + lax.axis_index("c")
    row0 = wid * ROWS_PER_W

    def chunk_body(ci, carry):
        r0 = row0 + ci * C
        pltpu.sync_copy(x_hbm.at[pl.ds(r0 * B, C * B)], xbuf)
        pltpu.sync_copy(pos_hbm.at[pl.ds(r0, C)], pbuf)

        def row_body(r, carry2):
            def g_body(g, carry3):
                col = g * LANES
                p = pbuf[r, pl.ds(col, LANES)]
                for b in range(B):
                    j = r * B + b
                    xbuf[j, pl.ds(col, LANES)] = xbuf[j, pl.ds(col, LANES)] + p
                return carry3
            return lax.fori_loop(0, G, g_body, carry2)

        lax.fori_loop(0, C, row_body, 0)
        pltpu.sync_copy(xbuf, out_hbm.at[pl.ds(r0 * B, C * B)])
        return carry

    lax.fori_loop(0, NCHUNK, chunk_body, 0)


def kernel(x, pos_table):
    x2 = x.reshape(T * B, D)
    out = _pos_add(x2, pos_table)
    return out.reshape(T, B, D)


# speedup vs baseline: 1.3997x; 1.3997x over previous
import functools

import jax
import jax.numpy as jnp
from jax import lax
from jax.experimental import pallas as pl
from jax.experimental.pallas import tpu as pltpu
from jax.experimental.pallas import tpu_sc as plsc

T = 8192
B = 4
D = 768
NC = 2
NS = 16
NW = NC * NS
ROWS_PER_W = T // NW
C = 8
NCHUNK = ROWS_PER_W // C
LANES = 16
G = D // LANES

_mesh = plsc.VectorSubcoreMesh(core_axis_name="c", subcore_axis_name="s")


@functools.partial(
    pl.kernel,
    mesh=_mesh,
    out_type=jax.ShapeDtypeStruct((T * B, D), jnp.float32),
    scratch_types=[
        pltpu.VMEM((C * B, D), jnp.float32),
        pltpu.VMEM((C * B, D), jnp.float32),
        pltpu.VMEM((C * B, D), jnp.float32),
        pltpu.VMEM((C * B, D), jnp.float32),
        pltpu.VMEM((C, D), jnp.float32),
        pltpu.VMEM((C, D), jnp.float32),
        pltpu.SemaphoreType.DMA,
        pltpu.SemaphoreType.DMA,
        pltpu.SemaphoreType.DMA,
        pltpu.SemaphoreType.DMA,
        pltpu.SemaphoreType.DMA,
        pltpu.SemaphoreType.DMA,
    ],
)
def _pos_add(x_hbm, pos_hbm, out_hbm,
             xbuf0, xbuf1, obuf0, obuf1, pbuf0, pbuf1,
             semx0, semx1, semp0, semp1, semo0, semo1):
    xbuf = (xbuf0, xbuf1)
    obuf = (obuf0, obuf1)
    pbuf = (pbuf0, pbuf1)
    semx = (semx0, semx1)
    semp = (semp0, semp1)
    semo = (semo0, semo1)

    wid = lax.axis_index("s") * NC + lax.axis_index("c")
    row0 = wid * ROWS_PER_W

    def x_copy(ci, slot):
        r0 = row0 + ci * C
        return pltpu.make_async_copy(
            x_hbm.at[pl.ds(r0 * B, C * B)], xbuf[slot], semx[slot])

    def p_copy(ci, slot):
        r0 = row0 + ci * C
        return pltpu.make_async_copy(
            pos_hbm.at[pl.ds(r0, C)], pbuf[slot], semp[slot])

    def o_copy(ci, slot):
        r0 = row0 + ci * C
        return pltpu.make_async_copy(
            obuf[slot], out_hbm.at[pl.ds(r0 * B, C * B)], semo[slot])

    def start_load(ci, slot):
        x_copy(ci, slot).start()
        p_copy(ci, slot).start()

    def compute(slot):
        xb, ob, pb = xbuf[slot], obuf[slot], pbuf[slot]

        def row_body(r, carry):
            @plsc.parallel_loop(0, G, unroll=4)
            def _g(g):
                col = g * LANES
                p = pb[r, pl.ds(col, LANES)]
                for b in range(B):
                    j = r * B + b
                    ob[j, pl.ds(col, LANES)] = xb[j, pl.ds(col, LANES)] + p
            return carry

        lax.fori_loop(0, C, row_body, 0)

    def process(ci, slot):
        @pl.when(ci + 1 < NCHUNK)
        def _():
            start_load(ci + 1, 1 - slot)

        x_copy(ci, slot).wait()
        p_copy(ci, slot).wait()

        @pl.when(ci >= 2)
        def _():
            o_copy(ci - 2, slot).wait()

        compute(slot)
        o_copy(ci, slot).start()

    start_load(0, 0)

    def pair_body(pi, carry):
        ci = pi * 2
        process(ci, 0)
        process(ci + 1, 1)
        return carry

    lax.fori_loop(0, NCHUNK // 2, pair_body, 0)

    o_copy(NCHUNK - 2, 0).wait()
    o_copy(NCHUNK - 1, 1).wait()


def kernel(x, pos_table):
    x2 = x.reshape(T * B, D)
    out = _pos_add(x2, pos_table)
    return out.reshape(T, B, D)


# speedup vs baseline: 4.3771x; 3.1271x over previous
import functools

import jax
import jax.numpy as jnp
from jax import lax
from jax.experimental import pallas as pl
from jax.experimental.pallas import tpu as pltpu
from jax.experimental.pallas import tpu_sc as plsc

T = 8192
B = 4
D = 768
NC = 2
NS = 16
NW = NC * NS
ROWS_PER_W = T // NW
C = 8
NCHUNK = ROWS_PER_W // C
LANES = 16
G = D // LANES

_mesh = plsc.VectorSubcoreMesh(core_axis_name="c", subcore_axis_name="s")


@functools.partial(
    pl.kernel,
    mesh=_mesh,
    out_type=jax.ShapeDtypeStruct((T, B, D), jnp.float32),
    scratch_types=[
        pltpu.VMEM((C, B, D), jnp.float32),
        pltpu.VMEM((C, B, D), jnp.float32),
        pltpu.VMEM((C, B, D), jnp.float32),
        pltpu.VMEM((C, B, D), jnp.float32),
        pltpu.VMEM((C, D), jnp.float32),
        pltpu.VMEM((C, D), jnp.float32),
        pltpu.SemaphoreType.DMA,
        pltpu.SemaphoreType.DMA,
        pltpu.SemaphoreType.DMA,
        pltpu.SemaphoreType.DMA,
        pltpu.SemaphoreType.DMA,
        pltpu.SemaphoreType.DMA,
    ],
)
def _pos_add(x_hbm, pos_hbm, out_hbm,
             xbuf0, xbuf1, obuf0, obuf1, pbuf0, pbuf1,
             semx0, semx1, semp0, semp1, semo0, semo1):
    xbuf = (xbuf0, xbuf1)
    obuf = (obuf0, obuf1)
    pbuf = (pbuf0, pbuf1)
    semx = (semx0, semx1)
    semp = (semp0, semp1)
    semo = (semo0, semo1)

    wid = lax.axis_index("s") * NC + lax.axis_index("c")
    row0 = wid * ROWS_PER_W

    def x_copy(ci, slot):
        r0 = row0 + ci * C
        return pltpu.make_async_copy(
            x_hbm.at[pl.ds(r0, C)], xbuf[slot], semx[slot])

    def p_copy(ci, slot):
        r0 = row0 + ci * C
        return pltpu.make_async_copy(
            pos_hbm.at[pl.ds(r0, C)], pbuf[slot], semp[slot])

    def o_copy(ci, slot):
        r0 = row0 + ci * C
        return pltpu.make_async_copy(
            obuf[slot], out_hbm.at[pl.ds(r0, C)], semo[slot])

    def start_load(ci, slot):
        x_copy(ci, slot).start()
        p_copy(ci, slot).start()

    def compute(slot):
        xb, ob, pb = xbuf[slot], obuf[slot], pbuf[slot]

        def row_body(r, carry):
            @plsc.parallel_loop(0, G, unroll=4)
            def _g(g):
                col = g * LANES
                p = pb[r, pl.ds(col, LANES)]
                for b in range(B):
                    ob[r, b, pl.ds(col, LANES)] = (
                        xb[r, b, pl.ds(col, LANES)] + p)
            return carry

        lax.fori_loop(0, C, row_body, 0)

    def process(ci, slot):
        @pl.when(ci + 1 < NCHUNK)
        def _():
            start_load(ci + 1, 1 - slot)

        x_copy(ci, slot).wait()
        p_copy(ci, slot).wait()

        @pl.when(ci >= 2)
        def _():
            o_copy(ci - 2, slot).wait()

        compute(slot)
        o_copy(ci, slot).start()

    start_load(0, 0)

    def pair_body(pi, carry):
        ci = pi * 2
        process(ci, 0)
        process(ci + 1, 1)
        return carry

    lax.fori_loop(0, NCHUNK // 2, pair_body, 0)

    o_copy(NCHUNK - 2, 0).wait()
    o_copy(NCHUNK - 1, 1).wait()


def kernel(x, pos_table):
    return _pos_add(x, pos_table)


# speedup vs baseline: 4.4039x; 1.0061x over previous
import functools

import jax
import jax.numpy as jnp
from jax import lax
from jax.experimental import pallas as pl
from jax.experimental.pallas import tpu as pltpu
from jax.experimental.pallas import tpu_sc as plsc

T = 8192
B = 4
D = 768
NC = 2
NS = 16
NW = NC * NS
ROWS_PER_W = T // NW
C = 8
LOG2C = C.bit_length() - 1
NCHUNK = ROWS_PER_W // C
LANES = 16
G = D // LANES

_mesh = plsc.VectorSubcoreMesh(core_axis_name="c", subcore_axis_name="s")


@functools.partial(
    pl.kernel,
    mesh=_mesh,
    out_type=jax.ShapeDtypeStruct((T, B, D), jnp.float32),
    scratch_types=[
        pltpu.VMEM((C, B, D), jnp.float32),
        pltpu.VMEM((C, B, D), jnp.float32),
        pltpu.VMEM((C, B, D), jnp.float32),
        pltpu.VMEM((C, B, D), jnp.float32),
        pltpu.VMEM((C, D), jnp.float32),
        pltpu.VMEM((C, D), jnp.float32),
        pltpu.SemaphoreType.DMA,
        pltpu.SemaphoreType.DMA,
        pltpu.SemaphoreType.DMA,
        pltpu.SemaphoreType.DMA,
        pltpu.SemaphoreType.DMA,
        pltpu.SemaphoreType.DMA,
    ],
)
def _pos_add(x_hbm, pos_hbm, out_hbm,
             xbuf0, xbuf1, obuf0, obuf1, pbuf0, pbuf1,
             semx0, semx1, semp0, semp1, semo0, semo1):
    xbuf = (xbuf0, xbuf1)
    obuf = (obuf0, obuf1)
    pbuf = (pbuf0, pbuf1)
    semx = (semx0, semx1)
    semp = (semp0, semp1)
    semo = (semo0, semo1)

    wid = lax.axis_index("s") * NC + lax.axis_index("c")
    row0 = wid * ROWS_PER_W

    def x_copy(ci, slot):
        r0 = row0 + ci * C
        return pltpu.make_async_copy(
            x_hbm.at[pl.ds(r0, C)], xbuf[slot], semx[slot])

    def p_copy(ci, slot):
        r0 = row0 + ci * C
        return pltpu.make_async_copy(
            pos_hbm.at[pl.ds(r0, C)], pbuf[slot], semp[slot])

    def o_copy(ci, slot):
        r0 = row0 + ci * C
        return pltpu.make_async_copy(
            obuf[slot], out_hbm.at[pl.ds(r0, C)], semo[slot])

    def start_load(ci, slot):
        x_copy(ci, slot).start()
        p_copy(ci, slot).start()

    def compute(slot):
        xb, ob, pb = xbuf[slot], obuf[slot], pbuf[slot]

        @plsc.parallel_loop(0, C * G, unroll=4)
        def _i(i):
            r = i & (C - 1)
            g = i >> LOG2C
            col = g * LANES
            p = pb[r, pl.ds(col, LANES)]
            for b in range(B):
                ob[r, b, pl.ds(col, LANES)] = (
                    xb[r, b, pl.ds(col, LANES)] + p)

    def process(ci, slot):
        @pl.when(ci + 1 < NCHUNK)
        def _():
            start_load(ci + 1, 1 - slot)

        x_copy(ci, slot).wait()
        p_copy(ci, slot).wait()

        @pl.when(ci >= 2)
        def _():
            o_copy(ci - 2, slot).wait()

        compute(slot)
        o_copy(ci, slot).start()

    start_load(0, 0)

    def pair_body(pi, carry):
        ci = pi * 2
        process(ci, 0)
        process(ci + 1, 1)
        return carry

    lax.fori_loop(0, NCHUNK // 2, pair_body, 0)

    o_copy(NCHUNK - 2, 0).wait()
    o_copy(NCHUNK - 1, 1).wait()


def kernel(x, pos_table):
    return _pos_add(x, pos_table)
